# Initial kernel scaffold; baseline (speedup 1.0000x reference)
import jax
import jax.numpy as jnp
from jax.experimental import pallas as pl


def kernel(x, W_gate, b_gate, W_experts, b_experts):
    raise NotImplementedError("write your pallas kernel here")


# speedup vs baseline: 22.0077x; 22.0077x over previous
import jax
import jax.numpy as jnp
from jax.experimental import pallas as pl
from jax.experimental.pallas import tpu as pltpu

_INTERPRET = False


def _gate_body(x_ref, wg_ref, bg_ref, idx_ref):
    logits = jnp.dot(x_ref[...], wg_ref[...], preferred_element_type=jnp.float32)
    logits = logits + bg_ref[...][None, :]
    m = jnp.max(logits, axis=-1, keepdims=True)
    p = jnp.exp(logits - m)
    p = p / jnp.sum(p, axis=-1, keepdims=True)
    pm = jnp.max(p, axis=-1, keepdims=True)
    lanes = jax.lax.broadcasted_iota(jnp.int32, p.shape, 1)
    idx_ref[...] = jnp.max(jnp.where(p >= pm, lanes, -1), axis=-1, keepdims=True)


def _dispatch_body(T, E, idx_ref, x_ref, we_ref, be_ref, out_ref, acc_ref):
    step = pl.program_id(0)

    @pl.when(step == 0)
    def _():
        xx = x_ref[...]
        for e in range(E):
            acc_ref[e] = (
                jnp.dot(xx, we_ref[e], preferred_element_type=jnp.float32)
                + be_ref[e][None, :]
            )

    for t in range(T):
        e = idx_ref[step * T + t]
        out_ref[pl.ds(t, 1)] = acc_ref[pl.ds(e, 1)]


def kernel(x, W_gate, b_gate, W_experts, b_experts):
    N, D_MODEL = x.shape
    E = W_gate.shape[1]
    D_FF = W_experts.shape[2]
    T = 8

    idx = pl.pallas_call(
        _gate_body,
        out_shape=jax.ShapeDtypeStruct((N, 1), jnp.int32),
        interpret=_INTERPRET,
    )(x, W_gate, b_gate)
    idx = idx.reshape(N)

    grid_spec = pltpu.PrefetchScalarGridSpec(
        num_scalar_prefetch=1,
        grid=(N // T,),
        in_specs=[
            pl.BlockSpec((N, D_MODEL), lambda i, idx_ref: (0, 0)),
            pl.BlockSpec((E, D_MODEL, D_FF), lambda i, idx_ref: (0, 0, 0)),
            pl.BlockSpec((E, D_FF), lambda i, idx_ref: (0, 0)),
        ],
        out_specs=pl.BlockSpec((T, N, D_FF), lambda i, idx_ref: (i, 0, 0)),
        scratch_shapes=[pltpu.VMEM((E, N, D_FF), jnp.float32)],
    )

    import functools
    out = pl.pallas_call(
        functools.partial(_dispatch_body, T, E),
        grid_spec=grid_spec,
        out_shape=jax.ShapeDtypeStruct((N, N, D_FF), jnp.float32),
        compiler_params=pltpu.CompilerParams(
            vmem_limit_bytes=128 * 1024 * 1024,
        ),
        interpret=_INTERPRET,
    )(idx, x, W_experts, b_experts)
    return out
